# baseline (device time: 88022 ns/iter reference)
import jax
import jax.numpy as jnp
from jax import lax
from jax.experimental import pallas as pl
from jax.experimental.pallas import tpu as pltpu

N_DEV = 4


def kernel(x, w_mat):
    m, k_per = x.shape
    _, n = w_mat.shape
    m_per = m // N_DEV

    def body(x_ref, w_ref, out_ref, send_ref, recv_ref, send_sems, recv_sems):
        my = lax.axis_index("i")
        left = lax.rem(my + N_DEV - 1, N_DEV)
        right = lax.rem(my + 1, N_DEV)

        barrier_sem = pltpu.get_barrier_semaphore()
        for nbr in [left, right]:
            pl.semaphore_signal(
                barrier_sem, inc=1,
                device_id=(nbr,), device_id_type=pl.DeviceIdType.MESH,
            )
        pl.semaphore_wait(barrier_sem, 2)

        def partial(c):
            xc = x_ref[pl.ds(c * m_per, m_per), :]
            return lax.dot_general(
                xc, w_ref[:, :], (((1,), (0,)), ((), ())),
                preferred_element_type=jnp.float32,
            )

        c0 = lax.rem(my + N_DEV - 1, N_DEV)
        send_ref[0] = partial(c0).astype(jnp.bfloat16)

        for s in range(N_DEV - 1):
            rdma = pltpu.make_async_remote_copy(
                src_ref=send_ref.at[s],
                dst_ref=recv_ref.at[s],
                send_sem=send_sems.at[s],
                recv_sem=recv_sems.at[s],
                device_id=(right,),
                device_id_type=pl.DeviceIdType.MESH,
            )
            rdma.start()
            rdma.wait()
            c = lax.rem(my + 2 * N_DEV - 2 - s, N_DEV)
            acc = recv_ref[s].astype(jnp.float32) + partial(c)
            if s < N_DEV - 2:
                send_ref[s + 1] = acc.astype(jnp.bfloat16)
            else:
                out_ref[:, :] = acc * jax.nn.sigmoid(acc)

    return pl.pallas_call(
        body,
        out_shape=jax.ShapeDtypeStruct((m_per, n), jnp.float32),
        in_specs=[
            pl.BlockSpec(memory_space=pltpu.VMEM),
            pl.BlockSpec(memory_space=pltpu.VMEM),
        ],
        out_specs=pl.BlockSpec(memory_space=pltpu.VMEM),
        scratch_shapes=[
            pltpu.VMEM((N_DEV - 1, m_per, n), jnp.bfloat16),
            pltpu.VMEM((N_DEV - 1, m_per, n), jnp.bfloat16),
            pltpu.SemaphoreType.DMA((N_DEV - 1,)),
            pltpu.SemaphoreType.DMA((N_DEV - 1,)),
        ],
        compiler_params=pltpu.CompilerParams(collective_id=0),
    )(x, w_mat)


# device time: 52261 ns/iter; 1.6843x vs baseline; 1.6843x over previous
import jax
import jax.numpy as jnp
from jax import lax
from jax.experimental import pallas as pl
from jax.experimental.pallas import tpu as pltpu

N_DEV = 4


def kernel(x, w_mat):
    m, k_per = x.shape
    _, n = w_mat.shape
    m_per = m // N_DEV
    h = n // 2

    def body(x_ref, w_ref, out_ref,
             send_p, recv_p, send_m, recv_m,
             ssem_p, rsem_p, ssem_m, rsem_m):
        my = lax.axis_index("i")
        left = lax.rem(my + N_DEV - 1, N_DEV)
        right = lax.rem(my + 1, N_DEV)

        barrier_sem = pltpu.get_barrier_semaphore()
        for nbr in [left, right]:
            pl.semaphore_signal(
                barrier_sem, inc=1,
                device_id=(nbr,), device_id_type=pl.DeviceIdType.MESH,
            )
        pl.semaphore_wait(barrier_sem, 2)

        def pmod(v):
            return lax.rem(v + 2 * N_DEV, N_DEV)

        def partial(c, lo):
            xc = x_ref[pl.ds(c * m_per, m_per), :]
            return lax.dot_general(
                xc, w_ref[:, lo:lo + h], (((1,), (0,)), ((), ())),
                preferred_element_type=jnp.float32,
            )

        send_p[0] = partial(pmod(my - 1), 0).astype(jnp.bfloat16)
        send_m[0] = partial(pmod(my + 1), h).astype(jnp.bfloat16)

        for s in range(N_DEV - 1):
            rp = pltpu.make_async_remote_copy(
                src_ref=send_p.at[s], dst_ref=recv_p.at[s],
                send_sem=ssem_p.at[s], recv_sem=rsem_p.at[s],
                device_id=(right,), device_id_type=pl.DeviceIdType.MESH,
            )
            rm = pltpu.make_async_remote_copy(
                src_ref=send_m.at[s], dst_ref=recv_m.at[s],
                send_sem=ssem_m.at[s], recv_sem=rsem_m.at[s],
                device_id=(left,), device_id_type=pl.DeviceIdType.MESH,
            )
            rp.start()
            rm.start()
            nxt_p = partial(pmod(my - 2 - s), 0)
            nxt_m = partial(pmod(my + 2 + s), h)
            rp.wait()
            acc_p = recv_p[s].astype(jnp.float32) + nxt_p
            if s < N_DEV - 2:
                send_p[s + 1] = acc_p.astype(jnp.bfloat16)
            rm.wait()
            acc_m = recv_m[s].astype(jnp.float32) + nxt_m
            if s < N_DEV - 2:
                send_m[s + 1] = acc_m.astype(jnp.bfloat16)
            else:
                out_ref[:, :h] = acc_p * jax.nn.sigmoid(acc_p)
                out_ref[:, h:] = acc_m * jax.nn.sigmoid(acc_m)

    comm = pltpu.VMEM((N_DEV - 1, m_per, h), jnp.bfloat16)
    sems = pltpu.SemaphoreType.DMA((N_DEV - 1,))
    return pl.pallas_call(
        body,
        out_shape=jax.ShapeDtypeStruct((m_per, n), jnp.float32),
        in_specs=[
            pl.BlockSpec(memory_space=pltpu.VMEM),
            pl.BlockSpec(memory_space=pltpu.VMEM),
        ],
        out_specs=pl.BlockSpec(memory_space=pltpu.VMEM),
        scratch_shapes=[comm, comm, comm, comm, sems, sems, sems, sems],
        compiler_params=pltpu.CompilerParams(collective_id=0),
    )(x, w_mat)


# device time: 46598 ns/iter; 1.8890x vs baseline; 1.1215x over previous
import jax
import jax.numpy as jnp
from jax import lax
from jax.experimental import pallas as pl
from jax.experimental.pallas import tpu as pltpu

N_DEV = 4
N_STRIPE = 2


def kernel(x, w_mat):
    m, k_per = x.shape
    _, n = w_mat.shape
    m_per = m // N_DEV
    w_s = n // (2 * N_STRIPE)

    def body(x_ref, w_ref, out_ref,
             send_p, recv_p, send_m, recv_m,
             ssem_p, rsem_p, ssem_m, rsem_m):
        my = lax.axis_index("i")
        left = lax.rem(my + N_DEV - 1, N_DEV)
        right = lax.rem(my + 1, N_DEV)

        barrier_sem = pltpu.get_barrier_semaphore()
        for nbr in [left, right]:
            pl.semaphore_signal(
                barrier_sem, inc=1,
                device_id=(nbr,), device_id_type=pl.DeviceIdType.MESH,
            )
        pl.semaphore_wait(barrier_sem, 2)

        def pmod(v):
            return lax.rem(v + 2 * N_DEV, N_DEV)

        def partial(c, lo):
            xc = x_ref[pl.ds(c * m_per, m_per), :]
            return lax.dot_general(
                xc, w_ref[:, lo:lo + w_s], (((1,), (0,)), ((), ())),
                preferred_element_type=jnp.float32,
            )

        def off_p(r):
            return r * w_s

        def off_m(r):
            return (N_STRIPE + r) * w_s

        def mk(r, s, plus):
            if plus:
                return pltpu.make_async_remote_copy(
                    src_ref=send_p.at[r, s], dst_ref=recv_p.at[r, s],
                    send_sem=ssem_p.at[r, s], recv_sem=rsem_p.at[r, s],
                    device_id=(right,), device_id_type=pl.DeviceIdType.MESH,
                )
            return pltpu.make_async_remote_copy(
                src_ref=send_m.at[r, s], dst_ref=recv_m.at[r, s],
                send_sem=ssem_m.at[r, s], recv_sem=rsem_m.at[r, s],
                device_id=(left,), device_id_type=pl.DeviceIdType.MESH,
            )

        for r in range(N_STRIPE):
            send_p[r, 0] = partial(pmod(my - 1), off_p(r)).astype(jnp.bfloat16)
            send_m[r, 0] = partial(pmod(my + 1), off_m(r)).astype(jnp.bfloat16)
            mk(r, 0, True).start()
            mk(r, 0, False).start()

        for s in range(N_DEV - 1):
            for r in range(N_STRIPE):
                nxt_p = partial(pmod(my - 2 - s), off_p(r))
                nxt_m = partial(pmod(my + 2 + s), off_m(r))
                mk(r, s, True).wait()
                acc_p = recv_p[r, s].astype(jnp.float32) + nxt_p
                if s < N_DEV - 2:
                    send_p[r, s + 1] = acc_p.astype(jnp.bfloat16)
                mk(r, s, False).wait()
                acc_m = recv_m[r, s].astype(jnp.float32) + nxt_m
                if s < N_DEV - 2:
                    send_m[r, s + 1] = acc_m.astype(jnp.bfloat16)
                    mk(r, s + 1, True).start()
                    mk(r, s + 1, False).start()
                else:
                    op, om = off_p(r), off_m(r)
                    out_ref[:, op:op + w_s] = acc_p * jax.nn.sigmoid(acc_p)
                    out_ref[:, om:om + w_s] = acc_m * jax.nn.sigmoid(acc_m)

    comm = pltpu.VMEM((N_STRIPE, N_DEV - 1, m_per, w_s), jnp.bfloat16)
    sems = pltpu.SemaphoreType.DMA((N_STRIPE, N_DEV - 1))
    return pl.pallas_call(
        body,
        out_shape=jax.ShapeDtypeStruct((m_per, n), jnp.float32),
        in_specs=[
            pl.BlockSpec(memory_space=pltpu.VMEM),
            pl.BlockSpec(memory_space=pltpu.VMEM),
        ],
        out_specs=pl.BlockSpec(memory_space=pltpu.VMEM),
        scratch_shapes=[comm, comm, comm, comm, sems, sems, sems, sems],
        compiler_params=pltpu.CompilerParams(collective_id=0),
    )(x, w_mat)
